# baseline (speedup 1.0000x reference)
import functools

import jax
import jax.numpy as jnp
from jax import lax
from jax.experimental import pallas as pl
from jax.experimental.pallas import tpu as pltpu
from jax.experimental.pallas import tpu_sc as plsc

N_FIELDS = 26
VOCAB = 100000
EMB = 16
N_CONT = 13
BATCH = 16384
OUT = 2
EMB_TOTAL = N_FIELDS * EMB
WINDOW = 128

ROWS_PER_FIELD = VOCAB * EMB // 128
ROWS_PER_FIELD_PAD = 12504
FIELD_STRIDE = ROWS_PER_FIELD_PAD * 8

CHUNKS = (8, 8, 8, 2)
WINDOWS_PER_FIELD = BATCH // WINDOW
N_WORKERS = 32


def _transpose_body(in_ref, i_ref, out_ref):
    x = in_ref[0]
    y = jnp.concatenate(
        [x[:, j * ROWS_PER_FIELD:(j + 1) * ROWS_PER_FIELD] for j in range(8)],
        axis=0)
    out_ref[0, :ROWS_PER_FIELD, :] = jax.lax.dot_general(
        y, i_ref[...], (((0,), (0,)), ((), ())),
        preferred_element_type=jnp.float32)


def _tc_transpose(tabT, ident, f0, nf):
    return pl.pallas_call(
        _transpose_body,
        grid=(nf,),
        in_specs=[
            pl.BlockSpec((1, EMB, VOCAB), lambda f: (f + f0, 0, 0)),
            pl.BlockSpec((128, 128), lambda f: (0, 0)),
        ],
        out_specs=pl.BlockSpec((1, ROWS_PER_FIELD_PAD, 128), lambda f: (f, 0, 0)),
        out_shape=jax.ShapeDtypeStruct((nf, ROWS_PER_FIELD_PAD, 128),
                                       jnp.float32),
    )(tabT, ident)


def _sc_gather(tables_flat, gidx2d, nf):
    mesh = plsc.VectorSubcoreMesh(core_axis_name="core", subcore_axis_name="subcore")
    n_windows = nf * WINDOWS_PER_FIELD
    w_per_tile = n_windows // N_WORKERS
    idx_per_tile = w_per_tile * WINDOW

    @functools.partial(
        pl.kernel,
        out_type=jax.ShapeDtypeStruct((BATCH, EMB * nf), jnp.float32),
        mesh=mesh,
        scratch_types=[
            pltpu.VMEM((w_per_tile, WINDOW), jnp.int32),
            pltpu.VMEM((idx_per_tile, EMB), jnp.float32),
            pltpu.SemaphoreType.DMA,
            pltpu.SemaphoreType.DMA,
        ],
        compiler_params=pltpu.CompilerParams(use_tc_tiling_on_sc=False),
    )
    def k(tab_hbm, idx_hbm, out_hbm, idx_v, rows_v, gsem, wsem):
        wid = lax.axis_index("subcore") * 2 + lax.axis_index("core")
        pltpu.sync_copy(idx_hbm.at[pl.ds(wid * w_per_tile, w_per_tile)], idx_v)

        @pl.loop(0, w_per_tile)
        def _fire(j):
            pltpu.async_copy(tab_hbm.at[idx_v.at[j]],
                             rows_v.at[pl.ds(j * WINDOW, WINDOW)], gsem)

        @pl.loop(0, w_per_tile)
        def _drain(j):
            pltpu.make_async_copy(tab_hbm.at[idx_v.at[j]],
                                  rows_v.at[pl.ds(j * WINDOW, WINDOW)], gsem).wait()
            w = wid * w_per_tile + j
            f = w // WINDOWS_PER_FIELD
            kk = w % WINDOWS_PER_FIELD
            pltpu.async_copy(
                rows_v.at[pl.ds(j * WINDOW, WINDOW)],
                out_hbm.at[pl.ds(kk * WINDOW, WINDOW), pl.ds(f * EMB, EMB)],
                wsem)

        @pl.loop(0, w_per_tile)
        def _drain_writes(j):
            w = wid * w_per_tile + j
            f = w // WINDOWS_PER_FIELD
            kk = w % WINDOWS_PER_FIELD
            pltpu.make_async_copy(
                rows_v.at[pl.ds(j * WINDOW, WINDOW)],
                out_hbm.at[pl.ds(kk * WINDOW, WINDOW), pl.ds(f * EMB, EMB)],
                wsem).wait()

    return k(tables_flat, gidx2d)


def _mlp_body(e0_ref, e1_ref, e2_ref, e3_ref, cT_ref, w0_ref, w1_ref, w2_ref,
              w3_ref, wc_ref, b1_ref, wh_ref, b2_ref, wo_ref, b3_ref, o_ref):
    h = jnp.dot(e0_ref[...], w0_ref[...], preferred_element_type=jnp.float32)
    h = h + jnp.dot(e1_ref[...], w1_ref[...], preferred_element_type=jnp.float32)
    h = h + jnp.dot(e2_ref[...], w2_ref[...], preferred_element_type=jnp.float32)
    h = h + jnp.dot(e3_ref[...], w3_ref[...], preferred_element_type=jnp.float32)
    h = h + jax.lax.dot_general(cT_ref[...], wc_ref[...], (((0,), (0,)), ((), ())),
                                preferred_element_type=jnp.float32)
    h = jnp.maximum(h + b1_ref[...], 0.0)
    h = jnp.dot(h, wh_ref[...], preferred_element_type=jnp.float32) + b2_ref[...]
    h = jnp.maximum(h, 0.0)
    o = jnp.dot(h, wo_ref[...], preferred_element_type=jnp.float32) + b3_ref[...]
    o_ref[...] = o.T


def _tc_mlp(es, x_cont, w1s, wc, b1, w2, b2, w3, b3):
    blk = 2048
    grid = BATCH // blk
    e_specs = [pl.BlockSpec((blk, 128), lambda i: (i, 0)) for _ in CHUNKS]
    w_specs = [pl.BlockSpec((128, 128), lambda i: (0, 0)) for _ in CHUNKS]
    return pl.pallas_call(
        _mlp_body,
        grid=(grid,),
        in_specs=e_specs + [pl.BlockSpec((N_CONT, blk), lambda i: (0, i))]
        + w_specs + [
            pl.BlockSpec((N_CONT, 128), lambda i: (0, 0)),
            pl.BlockSpec((1, 128), lambda i: (0, 0)),
            pl.BlockSpec((128, 64), lambda i: (0, 0)),
            pl.BlockSpec((1, 64), lambda i: (0, 0)),
            pl.BlockSpec((64, OUT), lambda i: (0, 0)),
            pl.BlockSpec((1, OUT), lambda i: (0, 0)),
        ],
        out_specs=pl.BlockSpec((OUT, blk), lambda i: (0, i)),
        out_shape=jax.ShapeDtypeStruct((OUT, BATCH), jnp.float32),
    )(*es, x_cont, *w1s, wc, b1, w2, b2, w3, b3)


def kernel(x_cat, x_cont, tables, W1, b1, W2, b2, W3, b3):
    xiT = x_cat.T.astype(jnp.int32)
    permT = (xiT % ROWS_PER_FIELD) * 8 + xiT // ROWS_PER_FIELD

    ident = jnp.eye(128, dtype=jnp.float32)
    tabT = jnp.transpose(tables, (0, 2, 1))

    es, w1s = [], []
    f0 = 0
    for nf in CHUNKS:
        loffs = jnp.arange(nf, dtype=jnp.int32) * FIELD_STRIDE
        gidx = permT[f0:f0 + nf] + loffs[:, None]
        w1c = W1[f0 * EMB:(f0 + nf) * EMB]
        if nf < 8:
            gidx = jnp.concatenate(
                [gidx, jnp.zeros((8 - nf, BATCH), jnp.int32)], axis=0)
            w1c = jnp.pad(w1c, ((0, (8 - nf) * EMB), (0, 0)))
        gidx = gidx.reshape(8 * WINDOWS_PER_FIELD, WINDOW)
        t = _tc_transpose(tabT, ident, f0, nf)
        es.append(_sc_gather(t.reshape(nf * FIELD_STRIDE, EMB), gidx, 8))
        w1s.append(w1c)
        f0 += nf

    x_contT = x_cont.T
    out_t = _tc_mlp(
        es, x_contT, w1s, W1[EMB_TOTAL:],
        b1.reshape(1, 128), W2, b2.reshape(1, 64), W3, b3.reshape(1, OUT),
    )
    return out_t.T


# speedup vs baseline: 3.3773x; 3.3773x over previous
import functools

import jax
import jax.numpy as jnp
from jax import lax
from jax.experimental import pallas as pl
from jax.experimental.pallas import tpu as pltpu
from jax.experimental.pallas import tpu_sc as plsc

N_FIELDS = 26
VOCAB = 100000
EMB = 16
N_CONT = 13
BATCH = 16384
OUT = 2
EMB_TOTAL = N_FIELDS * EMB
WINDOW = 128

ROWS_PER_FIELD = VOCAB * EMB // 128
ROWS_PER_FIELD_PAD = 12504
FIELD_STRIDE = ROWS_PER_FIELD_PAD * 8

CHUNKS = (8, 8, 8, 2)
WINDOWS_PER_FIELD = BATCH // WINDOW
N_WORKERS = 32


def _transpose_body(in_ref, i_ref, out_ref):
    x = in_ref[0]
    y = jnp.concatenate(
        [x[:, j * ROWS_PER_FIELD:(j + 1) * ROWS_PER_FIELD] for j in range(8)],
        axis=0)
    out_ref[0, :ROWS_PER_FIELD, :] = jax.lax.dot_general(
        y, i_ref[...], (((0,), (0,)), ((), ())),
        preferred_element_type=jnp.float32)


def _tc_transpose(tabT, ident, f0, nf):
    return pl.pallas_call(
        _transpose_body,
        grid=(nf,),
        in_specs=[
            pl.BlockSpec((1, EMB, VOCAB), lambda f: (f + f0, 0, 0)),
            pl.BlockSpec((128, 128), lambda f: (0, 0)),
        ],
        out_specs=pl.BlockSpec((1, ROWS_PER_FIELD_PAD, 128), lambda f: (f, 0, 0)),
        out_shape=jax.ShapeDtypeStruct((nf, ROWS_PER_FIELD_PAD, 128),
                                       jnp.float32),
    )(tabT, ident)


def _sc_gather(tables_flat, gidx2d, nf):
    mesh = plsc.VectorSubcoreMesh(core_axis_name="core", subcore_axis_name="subcore")
    n_windows = nf * WINDOWS_PER_FIELD
    w_per_tile = n_windows // N_WORKERS
    idx_per_tile = w_per_tile * WINDOW

    @functools.partial(
        pl.kernel,
        out_type=jax.ShapeDtypeStruct((BATCH, EMB * nf), jnp.float32),
        mesh=mesh,
        scratch_types=[
            pltpu.VMEM((w_per_tile, WINDOW), jnp.int32),
            pltpu.VMEM((idx_per_tile, EMB), jnp.float32),
            pltpu.SemaphoreType.DMA,
            pltpu.SemaphoreType.DMA,
        ],
        compiler_params=pltpu.CompilerParams(use_tc_tiling_on_sc=False),
    )
    def k(tab_hbm, idx_hbm, out_hbm, idx_v, rows_v, gsem, wsem):
        wid = lax.axis_index("subcore") * 2 + lax.axis_index("core")
        pltpu.sync_copy(idx_hbm.at[pl.ds(wid * w_per_tile, w_per_tile)], idx_v)

        @pl.loop(0, w_per_tile)
        def _fire(j):
            pltpu.async_copy(tab_hbm.at[idx_v.at[j]],
                             rows_v.at[pl.ds(j * WINDOW, WINDOW)], gsem)

        @pl.loop(0, w_per_tile)
        def _drain(j):
            pltpu.make_async_copy(tab_hbm.at[idx_v.at[j]],
                                  rows_v.at[pl.ds(j * WINDOW, WINDOW)], gsem).wait()
            w = wid * w_per_tile + j
            f = w // WINDOWS_PER_FIELD
            kk = w % WINDOWS_PER_FIELD
            pltpu.async_copy(
                rows_v.at[pl.ds(j * WINDOW, WINDOW)],
                out_hbm.at[pl.ds(kk * WINDOW, WINDOW), pl.ds(f * EMB, EMB)],
                wsem)

        @pl.loop(0, w_per_tile)
        def _drain_writes(j):
            w = wid * w_per_tile + j
            f = w // WINDOWS_PER_FIELD
            kk = w % WINDOWS_PER_FIELD
            pltpu.make_async_copy(
                rows_v.at[pl.ds(j * WINDOW, WINDOW)],
                out_hbm.at[pl.ds(kk * WINDOW, WINDOW), pl.ds(f * EMB, EMB)],
                wsem).wait()

    return k(tables_flat, gidx2d)


def _mlp_body(e0_ref, e1_ref, e2_ref, e3_ref, cT_ref, w0_ref, w1_ref, w2_ref,
              w3_ref, wc_ref, b1_ref, wh_ref, b2_ref, wo_ref, b3_ref, o_ref):
    h = jnp.dot(e0_ref[...], w0_ref[...], preferred_element_type=jnp.float32)
    h = h + jnp.dot(e1_ref[...], w1_ref[...], preferred_element_type=jnp.float32)
    h = h + jnp.dot(e2_ref[...], w2_ref[...], preferred_element_type=jnp.float32)
    h = h + jnp.dot(e3_ref[...], w3_ref[...], preferred_element_type=jnp.float32)
    h = h + jax.lax.dot_general(cT_ref[...], wc_ref[...], (((0,), (0,)), ((), ())),
                                preferred_element_type=jnp.float32)
    h = jnp.maximum(h + b1_ref[...], 0.0)
    h = jnp.dot(h, wh_ref[...], preferred_element_type=jnp.float32) + b2_ref[...]
    h = jnp.maximum(h, 0.0)
    o = jnp.dot(h, wo_ref[...], preferred_element_type=jnp.float32) + b3_ref[...]
    o_ref[...] = o.T


def _tc_mlp(es, x_cont, w1s, wc, b1, w2, b2, w3, b3):
    blk = 2048
    grid = BATCH // blk
    e_specs = [pl.BlockSpec((blk, 128), lambda i: (i, 0)) for _ in CHUNKS]
    w_specs = [pl.BlockSpec((128, 128), lambda i: (0, 0)) for _ in CHUNKS]
    return pl.pallas_call(
        _mlp_body,
        grid=(grid,),
        in_specs=e_specs + [pl.BlockSpec((N_CONT, blk), lambda i: (0, i))]
        + w_specs + [
            pl.BlockSpec((N_CONT, 128), lambda i: (0, 0)),
            pl.BlockSpec((1, 128), lambda i: (0, 0)),
            pl.BlockSpec((128, 64), lambda i: (0, 0)),
            pl.BlockSpec((1, 64), lambda i: (0, 0)),
            pl.BlockSpec((64, OUT), lambda i: (0, 0)),
            pl.BlockSpec((1, OUT), lambda i: (0, 0)),
        ],
        out_specs=pl.BlockSpec((OUT, blk), lambda i: (0, i)),
        out_shape=jax.ShapeDtypeStruct((OUT, BATCH), jnp.float32),
    )(*es, x_cont, *w1s, wc, b1, w2, b2, w3, b3)


def kernel(x_cat, x_cont, tables, W1, b1, W2, b2, W3, b3):
    xiT = x_cat.T.astype(jnp.int32)
    permT = (xiT % ROWS_PER_FIELD) * 8 + xiT // ROWS_PER_FIELD

    ident = jnp.eye(128, dtype=jnp.float32)
    tabT = jnp.transpose(tables, (0, 2, 1))

    es, w1s = [], []
    f0 = 0
    for nf in CHUNKS:
        loffs = jnp.arange(nf, dtype=jnp.int32) * FIELD_STRIDE
        gidx = permT[f0:f0 + nf] + loffs[:, None]
        w1c = W1[f0 * EMB:(f0 + nf) * EMB]
        if nf < 8:
            dummy = (jnp.arange(BATCH, dtype=jnp.int32) * 8) % (nf * FIELD_STRIDE)
            gidx = jnp.concatenate(
                [gidx, jnp.broadcast_to(dummy, (8 - nf, BATCH))], axis=0)
            w1c = jnp.pad(w1c, ((0, (8 - nf) * EMB), (0, 0)))
        gidx = gidx.reshape(8 * WINDOWS_PER_FIELD, WINDOW)
        t = _tc_transpose(tabT, ident, f0, nf)
        es.append(_sc_gather(t.reshape(nf * FIELD_STRIDE, EMB), gidx, 8))
        w1s.append(w1c)
        f0 += nf

    x_contT = x_cont.T
    out_t = _tc_mlp(
        es, x_contT, w1s, W1[EMB_TOTAL:],
        b1.reshape(1, 128), W2, b2.reshape(1, 64), W3, b3.reshape(1, OUT),
    )
    return out_t.T
